# speedup vs baseline: 17.1902x; 17.1902x over previous
import functools

import jax
import jax.numpy as jnp
from jax import lax
from jax.experimental import pallas as pl
from jax.experimental.pallas import tpu as pltpu
from jax.experimental.pallas import tpu_sc as plsc


RB = 256


def _topk_body(xr_ref, xa_ref, i0_ref, i1_ref, i2_ref):
    b = pl.program_id(0)
    rb = pl.program_id(1)
    xr = xr_ref[0]
    xa = xa_ref[0]
    n = xa.shape[0]

    x2r = jnp.sum(xr * xr, axis=1)
    x2a = jnp.sum(xa * xa, axis=1)
    g = lax.dot_general(xr, xa, (((1,), (1,)), ((), ())),
                        preferred_element_type=jnp.float32)
    d2 = x2r[:, None] + x2a[None, :] - 2.0 * g

    col = lax.broadcasted_iota(jnp.int32, (RB, n), 1)
    row_g = rb * RB + lax.broadcasted_iota(jnp.int32, (RB, n), 0)
    big = jnp.float32(3.0e38)
    d2 = jnp.where(col == row_g, big, d2)

    offs = b * n
    idxs = []
    for _ in range(3):
        m = jnp.min(d2, axis=1)
        amin = jnp.min(jnp.where(d2 == m[:, None], col, n), axis=1)
        idxs.append(amin + offs)
        d2 = jnp.where(col == amin[:, None], big, d2)

    i0_ref[0, 0, :] = idxs[0]
    i1_ref[0, 0, :] = idxs[1]
    i2_ref[0, 0, :] = idxs[2]


def _top3_indices(features):
    B, N, D = features.shape
    grid = (B, N // RB)
    out_sd = jax.ShapeDtypeStruct((B, 1, N), jnp.int32)
    return pl.pallas_call(
        _topk_body,
        grid=grid,
        in_specs=[
            pl.BlockSpec((1, RB, D), lambda b, r: (b, r, 0)),
            pl.BlockSpec((1, N, D), lambda b, r: (b, 0, 0)),
        ],
        out_specs=[
            pl.BlockSpec((1, 1, RB), lambda b, r: (b, 0, r)),
            pl.BlockSpec((1, 1, RB), lambda b, r: (b, 0, r)),
            pl.BlockSpec((1, 1, RB), lambda b, r: (b, 0, r)),
        ],
        out_shape=[out_sd, out_sd, out_sd],
    )(features, features)


_G = 32


def _make_sc_mix(BN, D, rows_per_w, info):
    NC = info.num_cores
    n_chunks = rows_per_w // _G
    mesh = plsc.VectorSubcoreMesh(core_axis_name="c", subcore_axis_name="s")

    @functools.partial(
        pl.kernel,
        mesh=mesh,
        out_type=jax.ShapeDtypeStruct((BN, D), jnp.float32),
        scratch_types=[
            pltpu.VMEM((_G,), jnp.int32),
            pltpu.VMEM((_G,), jnp.int32),
            pltpu.VMEM((_G,), jnp.int32),
            pltpu.VMEM((16,), jnp.float32),
            pltpu.VMEM((16,), jnp.float32),
            pltpu.VMEM((_G, D), jnp.float32),
            pltpu.VMEM((_G, D), jnp.float32),
            pltpu.VMEM((_G, D), jnp.float32),
            pltpu.VMEM((_G, D), jnp.float32),
            pltpu.SemaphoreType.DMA,
        ],
    )
    def sc_mix(feat_hbm, i0_hbm, i1_hbm, i2_hbm, ws_hbm, wn_hbm, out_hbm,
               i0_v, i1_v, i2_v, ws_v, wn_v, o_v, g0_v, g1_v, g2_v, sem):
        wid = lax.axis_index("s") * NC + lax.axis_index("c")
        base = wid * rows_per_w
        pltpu.sync_copy(ws_hbm, ws_v)
        pltpu.sync_copy(wn_hbm, wn_v)
        ws = ws_v[...]
        wn = wn_v[...]

        def chunk(c, carry):
            rbase = base + c * _G
            pltpu.sync_copy(i0_hbm.at[pl.ds(rbase, _G)], i0_v)
            pltpu.sync_copy(i1_hbm.at[pl.ds(rbase, _G)], i1_v)
            pltpu.sync_copy(i2_hbm.at[pl.ds(rbase, _G)], i2_v)
            cp_o = pltpu.async_copy(feat_hbm.at[pl.ds(rbase, _G)], o_v, sem)
            cp0 = pltpu.async_copy(feat_hbm.at[i0_v], g0_v, sem)
            cp1 = pltpu.async_copy(feat_hbm.at[i1_v], g1_v, sem)
            cp2 = pltpu.async_copy(feat_hbm.at[i2_v], g2_v, sem)
            cp_o.wait()
            cp0.wait()
            cp1.wait()
            cp2.wait()

            def row(r, acc_):
                def lane(i, __):
                    sl = pl.ds(i * 16, 16)
                    acc = g0_v[r, sl] + g1_v[r, sl] + g2_v[r, sl]
                    o_v[r, sl] = ws * o_v[r, sl] + wn * acc
                    return __
                return lax.fori_loop(0, D // 16, lane, acc_)

            lax.fori_loop(0, _G, row, 0)
            pltpu.sync_copy(o_v, out_hbm.at[pl.ds(rbase, _G)])
            return carry

        lax.fori_loop(0, n_chunks, chunk, 0)

    return sc_mix


def kernel(features, strength):
    B, N, D = features.shape
    BN = B * N
    info = plsc.get_sparse_core_info()
    NW = info.num_cores * info.num_subcores
    rows_per_w = BN // NW

    s = jnp.clip(strength, 0.0, 1.0)
    i0, i1, i2 = _top3_indices(features)
    feat_flat = features.reshape(BN, D)
    ws = jnp.full((16,), 1.0 - s, jnp.float32)
    wn = jnp.full((16,), s / 3.0, jnp.float32)

    sc_mix = _make_sc_mix(BN, D, rows_per_w, info)
    out = sc_mix(feat_flat, i0.reshape(BN), i1.reshape(BN), i2.reshape(BN),
                 ws, wn)
    return out.reshape(B, N, D)


# speedup vs baseline: 21.6050x; 1.2568x over previous
import functools

import jax
import jax.numpy as jnp
from jax import lax
from jax.experimental import pallas as pl
from jax.experimental.pallas import tpu as pltpu
from jax.experimental.pallas import tpu_sc as plsc


RB = 256


def _topk_body(xr_ref, xa_ref, i0_ref, i1_ref, i2_ref):
    b = pl.program_id(0)
    rb = pl.program_id(1)
    xr = xr_ref[0]
    xa = xa_ref[0]
    n = xa.shape[0]

    x2r = jnp.sum(xr * xr, axis=1)
    x2a = jnp.sum(xa * xa, axis=1)
    gm2 = lax.dot_general(xr * -2.0, xa, (((1,), (1,)), ((), ())),
                          preferred_element_type=jnp.float32)
    d2 = (x2r[:, None] + x2a[None, :]) + gm2
    d2 = jnp.sqrt(jnp.maximum(d2, 0.0))

    col = lax.broadcasted_iota(jnp.int32, (RB, n), 1)
    row = rb * RB + lax.broadcasted_iota(jnp.int32, (RB, n), 0)
    colf = col.astype(jnp.float32)
    big = jnp.float32(3.0e38)
    nf = jnp.float32(n)
    d2 = jnp.where(col == row, big, d2)

    offs = b * n
    outs = (i0_ref, i1_ref, i2_ref)
    for t in range(3):
        m = jnp.min(d2, axis=1)
        aminf = jnp.min(jnp.where(d2 == m[:, None], colf, nf), axis=1)
        outs[t][0, 0, :] = aminf.astype(jnp.int32) + offs
        if t < 2:
            d2 = jnp.where(colf == aminf[:, None], big, d2)


def _top3_indices(features):
    B, N, D = features.shape
    grid = (B, N // RB)
    out_sd = jax.ShapeDtypeStruct((B, 1, N), jnp.int32)
    return pl.pallas_call(
        _topk_body,
        grid=grid,
        in_specs=[
            pl.BlockSpec((1, RB, D), lambda b, r: (b, r, 0)),
            pl.BlockSpec((1, N, D), lambda b, r: (b, 0, 0)),
        ],
        out_specs=[
            pl.BlockSpec((1, 1, RB), lambda b, r: (b, 0, r)),
            pl.BlockSpec((1, 1, RB), lambda b, r: (b, 0, r)),
            pl.BlockSpec((1, 1, RB), lambda b, r: (b, 0, r)),
        ],
        out_shape=[out_sd, out_sd, out_sd],
    )(features, features)


_G = 16


def _make_sc_mix(BN, D, rows_per_w, info):
    NC = info.num_cores
    n_chunks = rows_per_w // _G
    n_pairs = n_chunks // 2
    mesh = plsc.VectorSubcoreMesh(core_axis_name="c", subcore_axis_name="s")
    groups = _G * D // 16

    @functools.partial(
        pl.kernel,
        mesh=mesh,
        out_type=jax.ShapeDtypeStruct((BN, D), jnp.float32),
        scratch_types=[
            pltpu.VMEM((rows_per_w,), jnp.int32),
            pltpu.VMEM((rows_per_w,), jnp.int32),
            pltpu.VMEM((rows_per_w,), jnp.int32),
            pltpu.VMEM((16,), jnp.float32),
            pltpu.VMEM((16,), jnp.float32),
            pltpu.VMEM((_G, D), jnp.float32),
            pltpu.VMEM((_G, D), jnp.float32),
            pltpu.VMEM((_G, D), jnp.float32),
            pltpu.VMEM((_G, D), jnp.float32),
            pltpu.VMEM((_G, D), jnp.float32),
            pltpu.VMEM((_G, D), jnp.float32),
            pltpu.VMEM((_G, D), jnp.float32),
            pltpu.VMEM((_G, D), jnp.float32),
            pltpu.VMEM((_G, D), jnp.float32),
            pltpu.VMEM((_G, D), jnp.float32),
            pltpu.SemaphoreType.DMA,
            pltpu.SemaphoreType.DMA,
            pltpu.SemaphoreType.DMA,
            pltpu.SemaphoreType.DMA,
        ],
    )
    def sc_mix(feat_hbm, i0_hbm, i1_hbm, i2_hbm, ws_hbm, wn_hbm, out_hbm,
               i0_v, i1_v, i2_v, ws_v, wn_v,
               oA, g0A, g1A, g2A, obA,
               oB, g0B, g1B, g2B, obB,
               inA, inB, outA, outB):
        wid = lax.axis_index("s") * NC + lax.axis_index("c")
        base = wid * rows_per_w
        pltpu.sync_copy(i0_hbm.at[pl.ds(base, rows_per_w)], i0_v)
        pltpu.sync_copy(i1_hbm.at[pl.ds(base, rows_per_w)], i1_v)
        pltpu.sync_copy(i2_hbm.at[pl.ds(base, rows_per_w)], i2_v)
        pltpu.sync_copy(ws_hbm, ws_v)
        pltpu.sync_copy(wn_hbm, wn_v)
        ws = ws_v[...]
        wn = wn_v[...]

        def issue_in(c, o_b, g0_b, g1_b, g2_b, sem):
            rbase = base + c * _G
            pltpu.async_copy(feat_hbm.at[pl.ds(rbase, _G)], o_b, sem)
            pltpu.async_copy(feat_hbm.at[i0_v[pl.ds(c * _G, _G)]], g0_b, sem)
            pltpu.async_copy(feat_hbm.at[i1_v[pl.ds(c * _G, _G)]], g1_b, sem)
            pltpu.async_copy(feat_hbm.at[i2_v[pl.ds(c * _G, _G)]], g2_b, sem)

        def wait_in(o_b, g0_b, g1_b, g2_b, sem):
            for buf in (o_b, g0_b, g1_b, g2_b):
                pltpu.make_async_copy(feat_hbm.at[pl.ds(0, _G)], buf, sem).wait()

        def compute(o_b, g0_b, g1_b, g2_b, ob_b):
            @plsc.parallel_loop(0, groups, 1, unroll=8)
            def _(i):
                r = lax.shift_right_logical(i, 5)
                sl = pl.ds((i & 31) * 16, 16)
                acc = g0_b[r, sl] + g1_b[r, sl] + g2_b[r, sl]
                ob_b[r, sl] = ws * o_b[r, sl] + wn * acc

        def issue_out(c, ob_b, sem):
            pltpu.async_copy(ob_b, out_hbm.at[pl.ds(base + c * _G, _G)], sem)

        def wait_out(ob_b, sem):
            pltpu.make_async_copy(ob_b, out_hbm.at[pl.ds(0, _G)], sem).wait()

        issue_in(0, oA, g0A, g1A, g2A, inA)
        issue_in(1, oB, g0B, g1B, g2B, inB)

        def pair(g, carry):
            cA = 2 * g
            cB = cA + 1
            wait_in(oA, g0A, g1A, g2A, inA)
            pl.when(g > 0)(lambda: wait_out(obA, outA))
            compute(oA, g0A, g1A, g2A, obA)
            issue_out(cA, obA, outA)
            pl.when(g < n_pairs - 1)(
                lambda: issue_in(cA + 2, oA, g0A, g1A, g2A, inA))

            wait_in(oB, g0B, g1B, g2B, inB)
            pl.when(g > 0)(lambda: wait_out(obB, outB))
            compute(oB, g0B, g1B, g2B, obB)
            issue_out(cB, obB, outB)
            pl.when(g < n_pairs - 1)(
                lambda: issue_in(cB + 2, oB, g0B, g1B, g2B, inB))
            return carry

        lax.fori_loop(0, n_pairs, pair, 0)
        wait_out(obA, outA)
        wait_out(obB, outB)

    return sc_mix


def kernel(features, strength):
    B, N, D = features.shape
    BN = B * N
    info = plsc.get_sparse_core_info()
    NW = info.num_cores * info.num_subcores
    rows_per_w = BN // NW

    s = jnp.clip(strength, 0.0, 1.0)
    i0, i1, i2 = _top3_indices(features)
    feat_flat = features.reshape(BN, D)
    ws = jnp.full((16,), 1.0 - s, jnp.float32)
    wn = jnp.full((16,), s / 3.0, jnp.float32)

    sc_mix = _make_sc_mix(BN, D, rows_per_w, info)
    out = sc_mix(feat_flat, i0.reshape(BN), i1.reshape(BN), i2.reshape(BN),
                 ws, wn)
    return out.reshape(B, N, D)
